# Initial kernel scaffold; baseline (speedup 1.0000x reference)
import functools

import jax
import jax.numpy as jnp
from jax import lax
from jax.experimental import pallas as pl
from jax.experimental.pallas import tpu as pltpu
from jax.experimental.pallas import tpu_sc as plsc

VOCAB = 100000
DIM = 128
BATCH = 4096
SEQ = 50

NC = 2
NS = 16
NW = NC * NS
B = BATCH * SEQ
BPW = B // NW
G = 128
NG = BPW // G


def _emb_body(idx_hbm, table_hbm, out_hbm, idx_v, buf_v, gsem):
    wid = lax.axis_index("s") * NC + lax.axis_index("c")
    row0 = wid * NG
    pltpu.sync_copy(idx_hbm.at[pl.ds(row0, NG)], idx_v)

    def body(g, carry):
        pltpu.async_copy(table_hbm.at[idx_v.at[g]], buf_v, gsem).wait()
        pltpu.sync_copy(buf_v, out_hbm.at[pl.ds((row0 + g) * G, G)])
        return carry

    lax.fori_loop(0, NG, body, 0)


def kernel(x, token_embedding):
    idx = x.reshape(B // G, G).astype(jnp.int32)
    mesh = plsc.VectorSubcoreMesh(core_axis_name="c", subcore_axis_name="s")
    grid_kernel = functools.partial(
        pl.kernel,
        out_type=jax.ShapeDtypeStruct((B, DIM), jnp.float32),
        mesh=mesh,
        scratch_types=[
            pltpu.VMEM((NG, G), jnp.int32),
            pltpu.VMEM((G, DIM), jnp.float32),
            pltpu.SemaphoreType.DMA,
        ],
    )
    out = grid_kernel(_emb_body)(idx, token_embedding)
    return out.reshape(BATCH, SEQ, DIM)


# speedup vs baseline: 2.9797x; 2.9797x over previous
import functools

import jax
import jax.numpy as jnp
from jax import lax
from jax.experimental import pallas as pl
from jax.experimental.pallas import tpu as pltpu
from jax.experimental.pallas import tpu_sc as plsc

VOCAB = 100000
DIM = 128
BATCH = 4096
SEQ = 50

NC = 2
NS = 16
NW = NC * NS
B = BATCH * SEQ
BPW = B // NW
G = 128
NG = BPW // G


def _emb_body(idx_hbm, table_hbm, out_hbm, idx_v, buf_v, gsem):
    wid = lax.axis_index("s") * NC + lax.axis_index("c")
    base = wid * BPW
    pltpu.sync_copy(idx_hbm.at[pl.ds(base, BPW)], idx_v)

    def body(g, carry):
        pltpu.async_copy(table_hbm.at[idx_v.at[pl.ds(g * G, G)]], buf_v, gsem).wait()
        pltpu.sync_copy(buf_v, out_hbm.at[pl.ds(base + g * G, G)])
        return carry

    lax.fori_loop(0, NG, body, 0)


def kernel(x, token_embedding):
    idx = x.reshape(B).astype(jnp.int32)
    mesh = plsc.VectorSubcoreMesh(core_axis_name="c", subcore_axis_name="s")
    grid_kernel = functools.partial(
        pl.kernel,
        out_type=jax.ShapeDtypeStruct((B, DIM), jnp.float32),
        mesh=mesh,
        scratch_types=[
            pltpu.VMEM((BPW,), jnp.int32),
            pltpu.VMEM((G, DIM), jnp.float32),
            pltpu.SemaphoreType.DMA,
        ],
    )
    out = grid_kernel(_emb_body)(idx, token_embedding)
    return out.reshape(BATCH, SEQ, DIM)


# speedup vs baseline: 3.3521x; 1.1250x over previous
import functools

import jax
import jax.numpy as jnp
from jax import lax
from jax.experimental import pallas as pl
from jax.experimental.pallas import tpu as pltpu
from jax.experimental.pallas import tpu_sc as plsc

VOCAB = 100000
DIM = 128
BATCH = 4096
SEQ = 50

NC = 2
NS = 16
NW = NC * NS
B = BATCH * SEQ
BPW = B // NW
G = 128
NG = BPW // G


NBUF = 5
LA = 3


def _emb_body(idx_hbm, table_hbm, out_hbm, idx_v, buf_v, gsem, ssem):
    wid = lax.axis_index("s") * NC + lax.axis_index("c")
    base = wid * BPW
    pltpu.sync_copy(idx_hbm.at[pl.ds(base, BPW)], idx_v)

    def start_gather(g, b):
        pltpu.async_copy(
            table_hbm.at[idx_v.at[pl.ds(g * G, G)]], buf_v.at[b], gsem.at[b])

    def wait_gather(g, b):
        pltpu.make_async_copy(
            table_hbm.at[idx_v.at[pl.ds(g * G, G)]], buf_v.at[b], gsem.at[b]
        ).wait()

    def start_store(g, b):
        pltpu.async_copy(
            buf_v.at[b], out_hbm.at[pl.ds(base + g * G, G)], ssem.at[b])

    def wait_store(g, b):
        pltpu.make_async_copy(
            buf_v.at[b], out_hbm.at[pl.ds(base + g * G, G)], ssem.at[b]
        ).wait()

    for g in range(LA):
        start_gather(g, g % NBUF)

    def step(j, carry):
        for b in range(NBUF):
            g = j * NBUF + b
            wait_gather(g, b)
            start_store(g, b)
            gl = g + LA
            bl = (b + LA) % NBUF

            @pl.when(gl < NG)
            def _():
                @pl.when(gl - NBUF >= 0)
                def _():
                    wait_store(gl - NBUF, bl)
                start_gather(gl, bl)
        return carry

    lax.fori_loop(0, NG // NBUF, step, 0)

    for g in range(NG - NBUF, NG):
        wait_store(g, g % NBUF)


def kernel(x, token_embedding):
    idx = x.reshape(B).astype(jnp.int32)
    mesh = plsc.VectorSubcoreMesh(core_axis_name="c", subcore_axis_name="s")
    grid_kernel = functools.partial(
        pl.kernel,
        out_type=jax.ShapeDtypeStruct((B, DIM), jnp.float32),
        mesh=mesh,
        scratch_types=[
            pltpu.VMEM((BPW,), jnp.int32),
            pltpu.VMEM((NBUF, G, DIM), jnp.float32),
            pltpu.SemaphoreType.DMA((NBUF,)),
            pltpu.SemaphoreType.DMA((NBUF,)),
        ],
    )
    out = grid_kernel(_emb_body)(idx, token_embedding)
    return out.reshape(BATCH, SEQ, DIM)


# speedup vs baseline: 5.9334x; 1.7701x over previous
import functools

import jax
import jax.numpy as jnp
from jax import lax
from jax.experimental import pallas as pl
from jax.experimental.pallas import tpu as pltpu
from jax.experimental.pallas import tpu_sc as plsc

VOCAB = 100000
DIM = 128
BATCH = 4096
SEQ = 50

NC = 2
NS = 16
NW = NC * NS
BPW = BATCH // NW
GB = 2
NGR = BPW // GB
NBUF = 4
LA = 2


def _emb_body(idx_hbm, table_hbm, out_hbm, idx_v, buf_v, gsem, ssem):
    wid = lax.axis_index("s") * NC + lax.axis_index("c")
    b0 = wid * BPW
    pltpu.sync_copy(idx_hbm.at[pl.ds(b0, BPW)], idx_v)

    def start_gather(g, b):
        for k in range(GB):
            pltpu.async_copy(
                table_hbm.at[idx_v.at[g * GB + k]], buf_v.at[b, k], gsem.at[b])

    def wait_gather(g, b):
        for k in range(GB):
            pltpu.make_async_copy(
                table_hbm.at[idx_v.at[g * GB + k]], buf_v.at[b, k], gsem.at[b]
            ).wait()

    def start_store(g, b):
        pltpu.async_copy(
            buf_v.at[b], out_hbm.at[pl.ds(b0 + g * GB, GB)], ssem.at[b])

    def wait_store(g, b):
        pltpu.make_async_copy(
            buf_v.at[b], out_hbm.at[pl.ds(b0 + g * GB, GB)], ssem.at[b]
        ).wait()

    for g in range(LA):
        start_gather(g, g % NBUF)

    def step(j, carry):
        for b in range(NBUF):
            g = j * NBUF + b
            wait_gather(g, b)
            start_store(g, b)
            gl = g + LA
            bl = (b + LA) % NBUF

            @pl.when(gl < NGR)
            def _():
                @pl.when(gl - NBUF >= 0)
                def _():
                    wait_store(gl - NBUF, bl)
                start_gather(gl, bl)
        return carry

    lax.fori_loop(0, NGR // NBUF, step, 0)

    for g in range(NGR - NBUF, NGR):
        wait_store(g, g % NBUF)


def kernel(x, token_embedding):
    idx = x.astype(jnp.int32)
    mesh = plsc.VectorSubcoreMesh(core_axis_name="c", subcore_axis_name="s")
    grid_kernel = functools.partial(
        pl.kernel,
        out_type=jax.ShapeDtypeStruct((BATCH, SEQ, DIM), jnp.float32),
        mesh=mesh,
        scratch_types=[
            pltpu.VMEM((BPW, SEQ), jnp.int32),
            pltpu.VMEM((NBUF, GB, SEQ, DIM), jnp.float32),
            pltpu.SemaphoreType.DMA((NBUF,)),
            pltpu.SemaphoreType.DMA((NBUF,)),
        ],
    )
    return grid_kernel(_emb_body)(idx, token_embedding)


# speedup vs baseline: 10.7402x; 1.8101x over previous
import functools

import jax
import jax.numpy as jnp
from jax import lax
from jax.experimental import pallas as pl
from jax.experimental.pallas import tpu as pltpu
from jax.experimental.pallas import tpu_sc as plsc

VOCAB = 100000
DIM = 128
BATCH = 4096
SEQ = 50

NC = 2
NS = 16
NW = NC * NS
BPW = BATCH // NW
NG = SEQ
NBUF = 5
LA = 3


def _emb_body(idx_hbm, table_hbm, out_hbm, idx_v, buf_v, gsem, ssem):
    wid = lax.axis_index("s") * NC + lax.axis_index("c")
    b0 = wid * BPW
    pltpu.sync_copy(idx_hbm.at[:, pl.ds(b0, BPW)], idx_v)

    def start_gather(g, b):
        pltpu.async_copy(table_hbm.at[idx_v.at[g]], buf_v.at[b], gsem.at[b])

    def wait_gather(g, b):
        pltpu.make_async_copy(
            table_hbm.at[idx_v.at[g]], buf_v.at[b], gsem.at[b]).wait()

    def start_store(g, b):
        pltpu.async_copy(
            buf_v.at[b], out_hbm.at[g, pl.ds(b0, BPW)], ssem.at[b])

    def wait_store(g, b):
        pltpu.make_async_copy(
            buf_v.at[b], out_hbm.at[g, pl.ds(b0, BPW)], ssem.at[b]).wait()

    for g in range(LA):
        start_gather(g, g % NBUF)

    def step(j, carry):
        for b in range(NBUF):
            g = j * NBUF + b
            wait_gather(g, b)
            start_store(g, b)
            gl = g + LA
            bl = (b + LA) % NBUF

            @pl.when(gl < NG)
            def _():
                @pl.when(gl - NBUF >= 0)
                def _():
                    wait_store(gl - NBUF, bl)
                start_gather(gl, bl)
        return carry

    lax.fori_loop(0, NG // NBUF, step, 0)

    for g in range(NG - NBUF, NG):
        wait_store(g, g % NBUF)


def kernel(x, token_embedding):
    idx = x.T.astype(jnp.int32)
    mesh = plsc.VectorSubcoreMesh(core_axis_name="c", subcore_axis_name="s")
    grid_kernel = functools.partial(
        pl.kernel,
        out_type=jax.ShapeDtypeStruct((SEQ, BATCH, DIM), jnp.float32),
        mesh=mesh,
        scratch_types=[
            pltpu.VMEM((SEQ, BPW), jnp.int32),
            pltpu.VMEM((NBUF, BPW, DIM), jnp.float32),
            pltpu.SemaphoreType.DMA((NBUF,)),
            pltpu.SemaphoreType.DMA((NBUF,)),
        ],
    )
    out = grid_kernel(_emb_body)(idx, token_embedding)
    return out.transpose(1, 0, 2)
